# baseline (speedup 1.0000x reference)
import functools
import jax
import jax.numpy as jnp
from jax import lax
from jax.experimental import pallas as pl
from jax.experimental.pallas import tpu as pltpu, tpu_sc as plsc


def _sc_copy_add(n, d):
    info = plsc.get_sparse_core_info()
    NC, NS = info.num_cores, info.num_subcores
    NW = NC * NS
    rows_w = (n // NW) // 8 * 8
    tail = n - NW * rows_w
    mesh = plsc.VectorSubcoreMesh(core_axis_name="c", subcore_axis_name="s")

    @functools.partial(
        pl.kernel,
        mesh=mesh,
        out_type=jax.ShapeDtypeStruct((n, d), jnp.float32),
        scratch_types=[
            pltpu.VMEM((2, d), jnp.float32),
            pltpu.VMEM((2, d), jnp.float32),
        ],
    )
    def k(x_hbm, v_hbm, out_hbm, row_v, val_v):
        wid = lax.axis_index("s") * NC + lax.axis_index("c")
        base = wid * rows_w
        pltpu.sync_copy(
            x_hbm.at[pl.ds(base, rows_w)], out_hbm.at[pl.ds(base, rows_w)]
        )

        if tail:
            @pl.when(wid == NW - 1)
            def _():
                pltpu.sync_copy(
                    x_hbm.at[pl.ds(NW * rows_w, tail)],
                    out_hbm.at[pl.ds(NW * rows_w, tail)],
                )

        @pl.when(wid == 0)
        def _():
            pltpu.sync_copy(x_hbm.at[pl.ds(0, 2)], row_v)
            pltpu.sync_copy(v_hbm, val_v)
            for r in range(2):
                for j in range(d // 16):
                    sl = pl.ds(j * 16, 16)
                    row_v[r, sl] = row_v[r, sl] + val_v[r, sl]
            pltpu.sync_copy(row_v, out_hbm.at[pl.ds(0, 2)])

    return k


def kernel(main_tensor, value):
    n, d = main_tensor.shape
    return _sc_copy_add(n, d)(main_tensor, value)


# speedup vs baseline: 15.2152x; 15.2152x over previous
import functools
import jax
import jax.numpy as jnp
from jax import lax
from jax.experimental import pallas as pl
from jax.experimental.pallas import tpu as pltpu, tpu_sc as plsc

_NBUF = 2
_CHUNK = 504


def _sc_copy_add(n, d):
    info = plsc.get_sparse_core_info()
    NC, NS = info.num_cores, info.num_subcores
    NW = NC * NS
    rows_w = (n // NW) // 8 * 8
    tail = n - NW * rows_w
    ngroups, rem = divmod(rows_w, _CHUNK * _NBUF)
    assert ngroups >= 2
    mesh = plsc.VectorSubcoreMesh(core_axis_name="c", subcore_axis_name="s")

    @functools.partial(
        pl.kernel,
        mesh=mesh,
        out_type=jax.ShapeDtypeStruct((n, d), jnp.float32),
        scratch_types=(
            [pltpu.VMEM((_CHUNK, d), jnp.float32) for _ in range(_NBUF)]
            + [pltpu.VMEM((8, d), jnp.float32), pltpu.VMEM((2, d), jnp.float32)]
            + [pltpu.SemaphoreType.DMA for _ in range(2 * _NBUF)]
        ),
    )
    def k(x_hbm, v_hbm, out_hbm, *refs):
        bufs = refs[:_NBUF]
        row_v, val_v = refs[_NBUF], refs[_NBUF + 1]
        sin = refs[_NBUF + 2 : _NBUF + 2 + _NBUF]
        sout = refs[_NBUF + 2 + _NBUF :]

        wid = lax.axis_index("s") * NC + lax.axis_index("c")
        base = wid * rows_w

        def in_cp(c, b):
            return pltpu.make_async_copy(
                x_hbm.at[pl.ds(base + c * _CHUNK, _CHUNK)], bufs[b], sin[b]
            )

        def out_cp(c, b):
            return pltpu.make_async_copy(
                bufs[b], out_hbm.at[pl.ds(base + c * _CHUNK, _CHUNK)], sout[b]
            )

        for b in range(_NBUF):
            in_cp(b, b).start()
        for b in range(_NBUF):
            in_cp(b, b).wait()
            out_cp(b, b).start()

        @pl.loop(1, ngroups)
        def _(g):
            c0 = g * _NBUF
            for b in range(_NBUF):
                out_cp(c0 - _NBUF + b, b).wait()
                in_cp(c0 + b, b).start()
            for b in range(_NBUF):
                in_cp(c0 + b, b).wait()
                out_cp(c0 + b, b).start()

        for b in range(_NBUF):
            out_cp((ngroups - 1) * _NBUF + b, b).wait()

        done = ngroups * _NBUF * _CHUNK
        r = rem
        off = done
        bi = 0
        while r > 0:
            step = min(r, _CHUNK)
            pltpu.sync_copy(
                x_hbm.at[pl.ds(base + off, step)],
                out_hbm.at[pl.ds(base + off, step)],
            )
            off += step
            r -= step
            bi += 1

        if tail:
            @pl.when(wid == NW - 1)
            def _():
                pltpu.sync_copy(
                    x_hbm.at[pl.ds(NW * rows_w, tail)],
                    out_hbm.at[pl.ds(NW * rows_w, tail)],
                )

        @pl.when(wid == 0)
        def _():
            pltpu.sync_copy(x_hbm.at[pl.ds(0, 8)], row_v)
            pltpu.sync_copy(v_hbm, val_v)
            for rr in range(2):
                for j in range(d // 16):
                    sl = pl.ds(j * 16, 16)
                    row_v[rr, sl] = row_v[rr, sl] + val_v[rr, sl]
            pltpu.sync_copy(row_v, out_hbm.at[pl.ds(0, 8)])

    return k


def kernel(main_tensor, value):
    n, d = main_tensor.shape
    return _sc_copy_add(n, d)(main_tensor, value)


# speedup vs baseline: 15.3351x; 1.0079x over previous
import functools
import jax
import jax.numpy as jnp
from jax import lax
from jax.experimental import pallas as pl
from jax.experimental.pallas import tpu as pltpu, tpu_sc as plsc

_NBUF = 2
_CHUNK = 504


def _sc_copy_add(n, d):
    info = plsc.get_sparse_core_info()
    NC, NS = info.num_cores, info.num_subcores
    NW = NC * NS
    nch = n // _CHUNK
    tail = n - nch * _CHUNK
    chunks_w, rem = divmod(nch, NW)
    ngroups, grem = divmod(chunks_w, _NBUF)
    assert ngroups >= 2 and rem == 0 and grem == 0
    mesh = plsc.VectorSubcoreMesh(core_axis_name="c", subcore_axis_name="s")

    @functools.partial(
        pl.kernel,
        mesh=mesh,
        out_type=jax.ShapeDtypeStruct((n, d), jnp.float32),
        scratch_types=(
            [pltpu.VMEM((_CHUNK, d), jnp.float32) for _ in range(_NBUF)]
            + [pltpu.VMEM((8, d), jnp.float32), pltpu.VMEM((2, d), jnp.float32)]
            + [pltpu.SemaphoreType.DMA for _ in range(2 * _NBUF)]
        ),
    )
    def k(x_hbm, v_hbm, out_hbm, *refs):
        bufs = refs[:_NBUF]
        row_v, val_v = refs[_NBUF], refs[_NBUF + 1]
        sin = refs[_NBUF + 2 : _NBUF + 2 + _NBUF]
        sout = refs[_NBUF + 2 + _NBUF :]

        wid = lax.axis_index("s") * NC + lax.axis_index("c")

        def _off(i):
            return (i * NW + wid) * _CHUNK

        def in_cp(c, b):
            return pltpu.make_async_copy(
                x_hbm.at[pl.ds(_off(c), _CHUNK)], bufs[b], sin[b]
            )

        def out_cp(c, b):
            return pltpu.make_async_copy(
                bufs[b], out_hbm.at[pl.ds(_off(c), _CHUNK)], sout[b]
            )

        for b in range(_NBUF):
            in_cp(b, b).start()
        for b in range(_NBUF):
            in_cp(b, b).wait()
            out_cp(b, b).start()

        @pl.loop(1, ngroups)
        def _(g):
            c0 = g * _NBUF
            for b in range(_NBUF):
                out_cp(c0 - _NBUF + b, b).wait()
                in_cp(c0 + b, b).start()
            for b in range(_NBUF):
                in_cp(c0 + b, b).wait()
                out_cp(c0 + b, b).start()

        for b in range(_NBUF):
            out_cp((ngroups - 1) * _NBUF + b, b).wait()


        if tail:
            @pl.when(wid == NW - 1)
            def _():
                pltpu.sync_copy(x_hbm.at[pl.ds(nch * _CHUNK, tail)], bufs[0].at[pl.ds(0, tail)])
                pltpu.sync_copy(bufs[0].at[pl.ds(0, tail)], out_hbm.at[pl.ds(nch * _CHUNK, tail)])

        @pl.when(wid == 0)
        def _():
            pltpu.sync_copy(x_hbm.at[pl.ds(0, 8)], row_v)
            pltpu.sync_copy(v_hbm, val_v)
            for rr in range(2):
                for j in range(d // 16):
                    sl = pl.ds(j * 16, 16)
                    row_v[rr, sl] = row_v[rr, sl] + val_v[rr, sl]
            pltpu.sync_copy(row_v, out_hbm.at[pl.ds(0, 8)])

    return k


def kernel(main_tensor, value):
    n, d = main_tensor.shape
    return _sc_copy_add(n, d)(main_tensor, value)
